# Initial kernel scaffold; baseline (speedup 1.0000x reference)
import jax
import jax.numpy as jnp
from jax.experimental import pallas as pl


def kernel(sequences):
    raise NotImplementedError("write your pallas kernel here")


# speedup vs baseline: 4.7184x; 4.7184x over previous
import functools

import jax
import jax.numpy as jnp
from jax import lax
from jax.experimental import pallas as pl
from jax.experimental.pallas import tpu as pltpu
from jax.experimental.pallas import tpu_sc as plsc

LANES = 16
NFEAT = 340
H4, H3, H2, H1 = 0, 256, 320, 336
O1, O2, O3, O4 = 0, 4, 20, 84


def _encoder_body(L, groups, seq_hbm, out_hbm, seq_v, hist_v, stage_v,
                  sem_in0, sem_in1, sem_out0, sem_out1):
    seq_blk = LANES * L
    iota = lax.iota(jnp.int32, LANES)
    lane_base = iota * NFEAT
    ones_f = jnp.full((LANES,), 1.0, jnp.float32)
    zeros_f = jnp.zeros((LANES,), jnp.float32)

    wid = lax.axis_index("s") * 2 + lax.axis_index("c")
    row0 = wid * (groups * LANES)

    sems_in = (sem_in0, sem_in1)
    sems_out = (sem_out0, sem_out1)

    def start_in(g, b):
        return pltpu.async_copy(
            seq_hbm.at[pl.ds((row0 + g * LANES) * L, seq_blk)],
            seq_v.at[pl.ds(b * seq_blk, seq_blk)],
            sems_in[b])

    in_copies = [start_in(0, 0), None]
    out_copies = [None, None]

    for g in range(groups):
        b = g & 1

        def zbody(i, _):
            hist_v[pl.ds(i * LANES, LANES)] = zeros_f
            return 0
        lax.fori_loop(0, NFEAT, zbody, 0)

        in_copies[b].wait()
        if g + 1 < groups:
            in_copies[1 - b] = start_in(g + 1, 1 - b)

        gidx = jnp.full((LANES,), b * seq_blk, jnp.int32) + iota * L
        c = jnp.zeros((LANES,), jnp.int32)
        for t in range(3):
            tok = plsc.load_gather(seq_v, [gidx + t])
            c = ((c << 2) & 255) + tok

        def mbody(t, carry):
            gi, cc = carry
            tok = plsc.load_gather(seq_v, [gi])
            cc = ((cc << 2) & 255) + tok
            plsc.addupdate_scatter(hist_v, [lane_base + cc], ones_f)
            return gi + 1, cc
        _, c = lax.fori_loop(3, L, mbody, (gidx + 3, c))

        plsc.addupdate_scatter(hist_v, [lane_base + (H3 + (c & 63))], ones_f)
        plsc.addupdate_scatter(hist_v, [lane_base + (H2 + (c & 15))], ones_f)
        plsc.addupdate_scatter(hist_v, [lane_base + (H1 + (c & 3))], ones_f)

        def marginalize(dst_off, src_off, n):
            def body(i, _):
                s = plsc.load_gather(hist_v, [lane_base + (src_off + 4 * i)])
                for j in range(1, 4):
                    s = s + plsc.load_gather(
                        hist_v, [lane_base + (src_off + 4 * i + j)])
                plsc.addupdate_scatter(hist_v, [lane_base + (dst_off + i)], s)
                return 0
            lax.fori_loop(0, n, body, 0)
        marginalize(H3, H4, 64)
        marginalize(H2, H3, 16)
        marginalize(H1, H2, 4)

        if out_copies[b] is not None:
            out_copies[b].wait()

        stage_lane = jnp.full((LANES,), b * LANES * NFEAT, jnp.int32) + lane_base

        def normalize(src_off, dst_off, n, inv):
            inv_v = jnp.full((LANES,), inv, jnp.float32)
            def body(i, _):
                v = plsc.load_gather(hist_v, [lane_base + (src_off + i)])
                plsc.store_scatter(
                    stage_v, [stage_lane + (dst_off + i)], v * inv_v)
                return 0
            lax.fori_loop(0, n, body, 0)
        normalize(H1, O1, 4, 1.0 / (L - 0))
        normalize(H2, O2, 16, 1.0 / (L - 1))
        normalize(H3, O3, 64, 1.0 / (L - 2))
        normalize(H4, O4, 256, 1.0 / (L - 3))

        out_copies[b] = pltpu.async_copy(
            stage_v.at[pl.ds(b * LANES * NFEAT, LANES * NFEAT)],
            out_hbm.at[pl.ds((row0 + g * LANES) * NFEAT, LANES * NFEAT)],
            sems_out[b])

    for cp in out_copies:
        if cp is not None:
            cp.wait()


def kernel(sequences):
    B, L = sequences.shape
    groups = B // (32 * LANES)
    mesh = plsc.VectorSubcoreMesh(core_axis_name="c", subcore_axis_name="s")
    run = pl.kernel(
        functools.partial(_encoder_body, L, groups),
        out_type=jax.ShapeDtypeStruct((B * NFEAT,), jnp.float32),
        mesh=mesh,
        scratch_types=[
            pltpu.VMEM((2 * LANES * L,), jnp.int32),
            pltpu.VMEM((LANES * NFEAT,), jnp.float32),
            pltpu.VMEM((2 * LANES * NFEAT,), jnp.float32),
            pltpu.SemaphoreType.DMA,
            pltpu.SemaphoreType.DMA,
            pltpu.SemaphoreType.DMA,
            pltpu.SemaphoreType.DMA,
        ],
        compiler_params=pltpu.CompilerParams(needs_layout_passes=False),
    )
    out = run(sequences.reshape(B * L))
    return out.reshape(B, NFEAT)
